# baseline (speedup 1.0000x reference)
import functools

import jax
import jax.numpy as jnp
from jax import lax
from jax.experimental import pallas as pl
from jax.experimental.pallas import tpu as pltpu
from jax.experimental.pallas import tpu_sc as plsc

EMBED_DIM = 16
BATCH = 16384
OUT_DIM = 3
N_PLAYERS = 1000000
N_OPENINGS = 1000

NC = 2
NS = 16
L = 16
NW = NC * NS
BPW = BATCH // NW

CW = 512
NB = 4
NCH = 62
RANGE = NCH * CW
LAST_LO = (NW - 1) * RANGE
LAST_NCH = (N_PLAYERS - LAST_LO) // CW
TAIL_LO = LAST_LO + LAST_NCH * CW
TAIL_REM = N_PLAYERS - TAIL_LO
MCAP = 1024
CCAP = 256
PLANE = OUT_DIM * BATCH


@functools.lru_cache(maxsize=1)
def _build_sc_call():
    mesh = plsc.VectorSubcoreMesh(core_axis_name="c", subcore_axis_name="s")

    @functools.partial(
        pl.kernel,
        out_type=jax.ShapeDtypeStruct((NW * PLANE,), jnp.float32),
        mesh=mesh,
        compiler_params=pltpu.CompilerParams(
            use_tc_tiling_on_sc=True, needs_layout_passes=False,
            disable_bounds_checks=True),
        scratch_types=[
            pltpu.VMEM((BATCH,), jnp.int32),
            pltpu.VMEM((BPW,), jnp.int32),
            pltpu.VMEM((BPW,), jnp.float32),
            pltpu.VMEM((EMBED_DIM, N_OPENINGS), jnp.float32),
            pltpu.VMEM((EMBED_DIM, CW), jnp.float32),
            pltpu.VMEM((EMBED_DIM, CW), jnp.float32),
            pltpu.VMEM((EMBED_DIM, CW), jnp.float32),
            pltpu.VMEM((EMBED_DIM, CW), jnp.float32),
            pltpu.VMEM((EMBED_DIM, TAIL_REM), jnp.float32),
            pltpu.VMEM((112,), jnp.float32),
            pltpu.VMEM((2 * MCAP,), jnp.int32),
            pltpu.VMEM((2 * MCAP,), jnp.int32),
            pltpu.VMEM((2 * CCAP,), jnp.int32),
            pltpu.VMEM((2 * CCAP,), jnp.int32),
            pltpu.VMEM((PLANE,), jnp.float32),
            pltpu.SemaphoreType.DMA,
            pltpu.SemaphoreType.DMA,
            pltpu.SemaphoreType.DMA,
            pltpu.SemaphoreType.DMA,
        ],
    )
    def sc_call(pid_hbm, oid_hbm, rat_hbm, ptab_hbm, ptail_hbm, otab_hbm,
                wb_hbm, out_hbm, pids_v, oid_v, rat_v, otab_v, buf0_v,
                buf1_v, buf2_v, buf3_v, tail_v, wb_v, mpid_v, mpos_v,
                cpid_v, cpos_v, plane_v, sem0, sem1, sem2, sem3):
        wid = lax.axis_index("c") * NS + lax.axis_index("s")
        base = pl.multiple_of(wid * BPW, BPW)
        lo = wid * RANGE
        hi = jnp.where(wid == NW - 1, N_PLAYERS, lo + RANGE)
        nch = jnp.where(wid == NW - 1, LAST_NCH, NCH)
        iota = lax.iota(jnp.int32, L)
        zeros = jnp.zeros((L,), jnp.float32)
        bufs = [buf0_v, buf1_v, buf2_v, buf3_v]
        sems = [sem0, sem1, sem2, sem3]
        descs = {}

        def fire(ci):
            clo = lo + ci * CW
            pltpu.async_copy(
                ptab_hbm.at[:, pl.ds(pl.multiple_of(clo, 128), CW)],
                bufs[ci % NB], sems[ci % NB])

        def wait_chunk(ci):
            pltpu.make_async_copy(
                ptab_hbm.at[:, pl.ds(0, CW)], bufs[ci % NB],
                sems[ci % NB]).wait()

        for _ci in range(NB):
            fire(_ci)

        pltpu.sync_copy(pid_hbm, pids_v)
        pltpu.sync_copy(oid_hbm.at[pl.ds(base, BPW)], oid_v)
        pltpu.sync_copy(rat_hbm.at[pl.ds(base, BPW)], rat_v)
        pltpu.sync_copy(otab_hbm, otab_v)
        pltpu.sync_copy(ptail_hbm, tail_v)
        pltpu.sync_copy(wb_hbm, wb_v)

        wvecs = [wb_v[pl.ds(k * L, L)] for k in range(112 // L)]

        def wsc(i):
            return wvecs[i // L][i % L]

        wo = [[wsc(d * OUT_DIM + j) for j in range(OUT_DIM)]
              for d in range(EMBED_DIM)]
        wp = [[wsc((EMBED_DIM + d) * OUT_DIM + j) for j in range(OUT_DIM)]
              for d in range(EMBED_DIM)]
        wr = [wsc(2 * EMBED_DIM * OUT_DIM + j) for j in range(OUT_DIM)]
        bb = [wsc((2 * EMBED_DIM + 1) * OUT_DIM + j) for j in range(OUT_DIM)]

        def zbody(i, carry):
            plane_v[pl.ds(i * L, L)] = zeros
            return carry

        lax.fori_loop(0, PLANE // L, zbody, 0)

        HB = BATCH // 2

        def mbody(i, carry):
            cnts = list(carry)
            for h in range(2):
                pvec = pids_v[pl.ds(h * HB + i * L, L)]
                m = (pvec >= lo) & (pvec < hi)
                plsc.store_compressed(
                    mpid_v.at[pl.ds(h * MCAP + cnts[h], L)], pvec, mask=m)
                plsc.store_compressed(
                    mpos_v.at[pl.ds(h * MCAP + cnts[h], L)],
                    h * HB + i * L + iota, mask=m)
                pc = plsc.all_reduce_population_count(m)[0]
                cnts[h] = jnp.minimum(cnts[h] + pc, MCAP - L)
            return tuple(cnts)

        cnts = lax.fori_loop(0, HB // L, mbody,
                             (jnp.int32(0), jnp.int32(0)))

        def obody(g, carry):
            sl = pl.ds(g * L, L)
            ovec = oid_v[sl]
            rat = rat_v[sl]
            acc = [rat * wr[j] + bb[j] for j in range(OUT_DIM)]
            for d in range(EMBED_DIM):
                dsp = jnp.full((L,), d, jnp.int32)
                odv = plsc.load_gather(otab_v, [dsp, ovec])
                for j in range(OUT_DIM):
                    acc[j] = acc[j] + odv * wo[d][j]
            for j in range(OUT_DIM):
                plane_v[pl.ds(j * BATCH + base + g * L, L)] = acc[j]
            return carry

        lax.fori_loop(0, BPW // L, obody, 0)

        def do_chunk(clo, width, buf):
            def rbody(g, carry):
                ccnts = list(carry)
                for h in range(2):
                    mvec = mpid_v[pl.ds(h * MCAP + g * L, L)]
                    posv = mpos_v[pl.ds(h * MCAP + g * L, L)]
                    valid = (g * L + iota) < cnts[h]
                    m = (mvec >= clo) & (mvec < clo + width) & valid
                    plsc.store_compressed(
                        cpid_v.at[pl.ds(h * CCAP + ccnts[h], L)], mvec, mask=m)
                    plsc.store_compressed(
                        cpos_v.at[pl.ds(h * CCAP + ccnts[h], L)], posv, mask=m)
                    pc = plsc.all_reduce_population_count(m)[0]
                    ccnts[h] = jnp.minimum(ccnts[h] + pc, CCAP - L)
                return tuple(ccnts)

            nr = (jnp.maximum(cnts[0], cnts[1]) + L - 1) // L
            ccnts = lax.fori_loop(0, nr, rbody, (jnp.int32(0), jnp.int32(0)))

            def make_pbody(h):
                def pbody(g, carry):
                    sl = pl.ds(g * L, L)
                    jvec = jnp.clip(cpid_v[pl.ds(h * CCAP + g * L, L)] - clo, 0, width - 1)
                    posv = cpos_v[pl.ds(h * CCAP + g * L, L)]
                    mask = (g * L + iota) < ccnts[h]
                    acc = [zeros for _ in range(OUT_DIM)]
                    for d in range(EMBED_DIM):
                        dsp = jnp.full((L,), d, jnp.int32)
                        pdv = plsc.load_gather(buf, [dsp, jvec])
                        for j in range(OUT_DIM):
                            acc[j] = acc[j] + pdv * wp[d][j]
                    for j in range(OUT_DIM):
                        plsc.addupdate_scatter(
                            plane_v, [posv + j * BATCH], acc[j], mask=mask)
                    return carry
                return pbody

            for h in range(2):
                lax.fori_loop(0, (ccnts[h] + L - 1) // L, make_pbody(h), 0)

        def quad_body(k, carry):
            for par in range(NB):
                ci = NB * k + par

                def step(ci=ci, par=par):
                    wait_chunk(par)
                    do_chunk(lo + ci * CW, CW, bufs[par])
                    pl.when(ci + NB < nch)(lambda: fire_dyn(ci + NB, par))

                pl.when(ci < nch)(step)
            return carry

        def fire_dyn(ci, par):
            clo = lo + ci * CW
            pltpu.async_copy(
                ptab_hbm.at[:, pl.ds(pl.multiple_of(clo, 128), CW)],
                bufs[par], sems[par])

        lax.fori_loop(0, (NCH + NB - 1) // NB, quad_body, 0)

        @pl.when(wid == NW - 1)
        def _last():
            do_chunk(TAIL_LO, TAIL_REM, tail_v)

        pltpu.sync_copy(
            plane_v, out_hbm.at[pl.ds(pl.multiple_of(wid * PLANE, 1024),
                                      PLANE)])

    return sc_call


@functools.lru_cache(maxsize=1)
def _build_tc_sum():
    def body(planes_ref, out_ref, acc_ref, sem):
        cps = [pltpu.make_async_copy(
            planes_ref.at[pl.ds(w * PLANE, PLANE)], acc_ref.at[w], sem)
            for w in range(NW)]
        for c in cps:
            c.start()
        for c in cps:
            c.wait()
        tot = acc_ref[0]
        for w in range(1, NW):
            tot = tot + acc_ref[w]
        out_ref[...] = tot

    return pl.pallas_call(
        body,
        in_specs=[pl.BlockSpec(memory_space=pl.ANY)],
        out_shape=jax.ShapeDtypeStruct((PLANE,), jnp.float32),
        scratch_shapes=[
            pltpu.VMEM((NW, PLANE), jnp.float32),
            pltpu.SemaphoreType.DMA,
        ],
    )


def kernel(opening_input, player_input, rating_input, player_table,
           opening_table, W, b):
    oid = opening_input.reshape(-1).astype(jnp.int32)
    pid = player_input.reshape(-1).astype(jnp.int32)
    wb = jnp.zeros((112,), jnp.float32)
    wb = wb.at[:(2 * EMBED_DIM + 1) * OUT_DIM].set(
        W.astype(jnp.float32).reshape(-1))
    wb = wb.at[(2 * EMBED_DIM + 1) * OUT_DIM:
               (2 * EMBED_DIM + 2) * OUT_DIM].set(b.astype(jnp.float32))
    ptail = player_table[TAIL_LO:].T
    planes = _build_sc_call()(pid, oid, rating_input.astype(jnp.float32),
                              player_table.T, ptail, opening_table.T, wb)
    summed = _build_tc_sum()(planes)
    return summed.reshape(OUT_DIM, BATCH).T


# speedup vs baseline: 1.1851x; 1.1851x over previous
import functools

import jax
import jax.numpy as jnp
from jax import lax
from jax.experimental import pallas as pl
from jax.experimental.pallas import tpu as pltpu
from jax.experimental.pallas import tpu_sc as plsc

EMBED_DIM = 16
BATCH = 16384
OUT_DIM = 3
N_PLAYERS = 1000000
N_OPENINGS = 1000

NC = 2
NS = 16
L = 16
NW = NC * NS
BPW = BATCH // NW

CW = 1024
NCH = 31
RANGE = NCH * CW
LAST_LO = (NW - 1) * RANGE
LAST_NCH = (N_PLAYERS - LAST_LO) // CW
LAST_REM_LO = LAST_LO + LAST_NCH * CW
LAST_REM = 512
TAIL_LO = LAST_REM_LO + LAST_REM
TAIL_REM = N_PLAYERS - TAIL_LO
MCAP = 1024
CCAP = 256
PLANE = OUT_DIM * BATCH


@functools.lru_cache(maxsize=1)
def _build_sc_call():
    mesh = plsc.VectorSubcoreMesh(core_axis_name="c", subcore_axis_name="s")

    @functools.partial(
        pl.kernel,
        out_type=jax.ShapeDtypeStruct((NW * PLANE,), jnp.float32),
        mesh=mesh,
        compiler_params=pltpu.CompilerParams(
            use_tc_tiling_on_sc=True, needs_layout_passes=False,
            disable_bounds_checks=True),
        scratch_types=[
            pltpu.VMEM((BATCH,), jnp.int32),
            pltpu.VMEM((BPW,), jnp.int32),
            pltpu.VMEM((BPW,), jnp.float32),
            pltpu.VMEM((EMBED_DIM, N_OPENINGS), jnp.float32),
            pltpu.VMEM((EMBED_DIM, CW), jnp.float32),
            pltpu.VMEM((EMBED_DIM, CW), jnp.float32),
            pltpu.VMEM((EMBED_DIM, TAIL_REM), jnp.float32),
            pltpu.VMEM((112,), jnp.float32),
            pltpu.VMEM((2 * MCAP,), jnp.int32),
            pltpu.VMEM((2 * MCAP,), jnp.int32),
            pltpu.VMEM((2 * CCAP,), jnp.int32),
            pltpu.VMEM((2 * CCAP,), jnp.int32),
            pltpu.VMEM((PLANE,), jnp.float32),
            pltpu.SemaphoreType.DMA,
            pltpu.SemaphoreType.DMA,
        ],
    )
    def sc_call(pid_hbm, oid_hbm, rat_hbm, ptab_hbm, ptail_hbm, otab_hbm,
                wb_hbm, out_hbm, pids_v, oid_v, rat_v, otab_v, buf0_v,
                buf1_v, tail_v, wb_v, mpid_v, mpos_v, cpid_v, cpos_v,
                plane_v, sem0, sem1):
        wid = lax.axis_index("c") * NS + lax.axis_index("s")
        base = pl.multiple_of(wid * BPW, BPW)
        lo = wid * RANGE
        hi = jnp.where(wid == NW - 1, N_PLAYERS, lo + RANGE)
        nch = jnp.where(wid == NW - 1, LAST_NCH, NCH)
        iota = lax.iota(jnp.int32, L)
        zeros = jnp.zeros((L,), jnp.float32)
        bufs = [buf0_v, buf1_v]
        sems = [sem0, sem1]
        descs = {}

        def fire(ci):
            clo = lo + ci * CW
            pltpu.async_copy(
                ptab_hbm.at[:, pl.ds(pl.multiple_of(clo, 128), CW)],
                bufs[ci % 2], sems[ci % 2])

        def wait_chunk(ci):
            pltpu.make_async_copy(
                ptab_hbm.at[:, pl.ds(0, CW)], bufs[ci % 2],
                sems[ci % 2]).wait()

        fire(0)
        fire(1)

        pltpu.sync_copy(pid_hbm, pids_v)
        pltpu.sync_copy(oid_hbm.at[pl.ds(base, BPW)], oid_v)
        pltpu.sync_copy(rat_hbm.at[pl.ds(base, BPW)], rat_v)
        pltpu.sync_copy(otab_hbm, otab_v)
        pltpu.sync_copy(ptail_hbm, tail_v)
        pltpu.sync_copy(wb_hbm, wb_v)

        wvecs = [wb_v[pl.ds(k * L, L)] for k in range(112 // L)]

        def wsc(i):
            return wvecs[i // L][i % L]

        wo = [[wsc(d * OUT_DIM + j) for j in range(OUT_DIM)]
              for d in range(EMBED_DIM)]
        wp = [[wsc((EMBED_DIM + d) * OUT_DIM + j) for j in range(OUT_DIM)]
              for d in range(EMBED_DIM)]
        wr = [wsc(2 * EMBED_DIM * OUT_DIM + j) for j in range(OUT_DIM)]
        bb = [wsc((2 * EMBED_DIM + 1) * OUT_DIM + j) for j in range(OUT_DIM)]

        def zbody(i, carry):
            plane_v[pl.ds(i * L, L)] = zeros
            return carry

        lax.fori_loop(0, PLANE // L, zbody, 0)

        HB = BATCH // 2

        def mbody(i, carry):
            cnts = list(carry)
            for h in range(2):
                pvec = pids_v[pl.ds(h * HB + i * L, L)]
                m = (pvec >= lo) & (pvec < hi)
                plsc.store_compressed(
                    mpid_v.at[pl.ds(h * MCAP + cnts[h], L)], pvec, mask=m)
                plsc.store_compressed(
                    mpos_v.at[pl.ds(h * MCAP + cnts[h], L)],
                    h * HB + i * L + iota, mask=m)
                pc = plsc.all_reduce_population_count(m)[0]
                cnts[h] = jnp.minimum(cnts[h] + pc, MCAP - L)
            return tuple(cnts)

        cnts = lax.fori_loop(0, HB // L, mbody,
                             (jnp.int32(0), jnp.int32(0)))

        def obody(g, carry):
            sl = pl.ds(g * L, L)
            ovec = oid_v[sl]
            rat = rat_v[sl]
            acc = [rat * wr[j] + bb[j] for j in range(OUT_DIM)]
            for d in range(EMBED_DIM):
                dsp = jnp.full((L,), d, jnp.int32)
                odv = plsc.load_gather(otab_v, [dsp, ovec])
                for j in range(OUT_DIM):
                    acc[j] = acc[j] + odv * wo[d][j]
            for j in range(OUT_DIM):
                plane_v[pl.ds(j * BATCH + base + g * L, L)] = acc[j]
            return carry

        lax.fori_loop(0, BPW // L, obody, 0)

        def do_chunk(clo, width, buf):
            def rbody(g, carry):
                ccnts = list(carry)
                for h in range(2):
                    mvec = mpid_v[pl.ds(h * MCAP + g * L, L)]
                    posv = mpos_v[pl.ds(h * MCAP + g * L, L)]
                    valid = (g * L + iota) < cnts[h]
                    m = (mvec >= clo) & (mvec < clo + width) & valid
                    plsc.store_compressed(
                        cpid_v.at[pl.ds(h * CCAP + ccnts[h], L)], mvec, mask=m)
                    plsc.store_compressed(
                        cpos_v.at[pl.ds(h * CCAP + ccnts[h], L)], posv, mask=m)
                    pc = plsc.all_reduce_population_count(m)[0]
                    ccnts[h] = jnp.minimum(ccnts[h] + pc, CCAP - L)
                return tuple(ccnts)

            nr = (jnp.maximum(cnts[0], cnts[1]) + L - 1) // L
            ccnts = lax.fori_loop(0, nr, rbody, (jnp.int32(0), jnp.int32(0)))

            def make_pbody(h):
                def pbody(g, carry):
                    sl = pl.ds(g * L, L)
                    jvec = jnp.clip(cpid_v[pl.ds(h * CCAP + g * L, L)] - clo, 0, width - 1)
                    posv = cpos_v[pl.ds(h * CCAP + g * L, L)]
                    mask = (g * L + iota) < ccnts[h]
                    acc = [zeros for _ in range(OUT_DIM)]
                    for d in range(EMBED_DIM):
                        dsp = jnp.full((L,), d, jnp.int32)
                        pdv = plsc.load_gather(buf, [dsp, jvec])
                        for j in range(OUT_DIM):
                            acc[j] = acc[j] + pdv * wp[d][j]
                    for j in range(OUT_DIM):
                        plsc.addupdate_scatter(
                            plane_v, [posv + j * BATCH], acc[j], mask=mask)
                    return carry
                return pbody

            for h in range(2):
                lax.fori_loop(0, (ccnts[h] + L - 1) // L, make_pbody(h), 0)

        def pair_body(k, carry):
            for par in range(2):
                ci = 2 * k + par

                def step(ci=ci, par=par):
                    wait_chunk(par)
                    do_chunk(lo + ci * CW, CW, bufs[par])
                    pl.when(ci + 2 < nch)(lambda: fire_dyn(ci + 2, par))

                pl.when(ci < nch)(step)
            return carry

        def fire_dyn(ci, par):
            clo = lo + ci * CW
            pltpu.async_copy(
                ptab_hbm.at[:, pl.ds(pl.multiple_of(clo, 128), CW)],
                bufs[par], sems[par])

        lax.fori_loop(0, (NCH + 1) // 2, pair_body, 0)

        @pl.when(wid == NW - 1)
        def _last():
            pltpu.sync_copy(
                ptab_hbm.at[:, pl.ds(LAST_REM_LO, LAST_REM)],
                buf0_v.at[:, pl.ds(0, LAST_REM)])
            do_chunk(LAST_REM_LO, LAST_REM, buf0_v)
            do_chunk(TAIL_LO, TAIL_REM, tail_v)

        pltpu.sync_copy(
            plane_v, out_hbm.at[pl.ds(pl.multiple_of(wid * PLANE, 1024),
                                      PLANE)])

    return sc_call


@functools.lru_cache(maxsize=1)
def _build_tc_sum():
    def body(planes_ref, out_ref, acc_ref, sem):
        cps = [pltpu.make_async_copy(
            planes_ref.at[pl.ds(w * PLANE, PLANE)], acc_ref.at[w], sem)
            for w in range(NW)]
        for c in cps:
            c.start()
        for c in cps:
            c.wait()
        tot = acc_ref[0]
        for w in range(1, NW):
            tot = tot + acc_ref[w]
        out_ref[...] = tot

    return pl.pallas_call(
        body,
        in_specs=[pl.BlockSpec(memory_space=pl.ANY)],
        out_shape=jax.ShapeDtypeStruct((PLANE,), jnp.float32),
        scratch_shapes=[
            pltpu.VMEM((NW, PLANE), jnp.float32),
            pltpu.SemaphoreType.DMA,
        ],
    )


def kernel(opening_input, player_input, rating_input, player_table,
           opening_table, W, b):
    oid = opening_input.reshape(-1).astype(jnp.int32)
    pid = player_input.reshape(-1).astype(jnp.int32)
    wb = jnp.zeros((112,), jnp.float32)
    wb = wb.at[:(2 * EMBED_DIM + 1) * OUT_DIM].set(
        W.astype(jnp.float32).reshape(-1))
    wb = wb.at[(2 * EMBED_DIM + 1) * OUT_DIM:
               (2 * EMBED_DIM + 2) * OUT_DIM].set(b.astype(jnp.float32))
    ptail = player_table[TAIL_LO:].T
    planes = _build_sc_call()(pid, oid, rating_input.astype(jnp.float32),
                              player_table.T, ptail, opening_table.T, wb)
    summed = _build_tc_sum()(planes)
    return summed.reshape(OUT_DIM, BATCH).T
